# baseline (speedup 1.0000x reference)
import jax
import jax.numpy as jnp
from jax import lax
from jax.experimental import pallas as pl
from jax.experimental.pallas import tpu as pltpu
from jax.experimental.pallas import tpu_sc as plsc

N = 10000
F = 256
H = 4
E = 160000
STEPS = 4
DT = 0.25

NC = 2
NS = 16
NW = NC * NS
L = 16

EPAD = 163840
EW = EPAD // NW
CH = 128
NCH = EW // CH

ETILE = EPAD // NS
DC = 64
NDC = ETILE // DC

NHALF = N // NC
SROWS = 5120
DUMMY = NHALF

DN = 10240
DTILE = DN // NS
DNF = DN * H

BN = 512
GN = (N + BN - 1) // BN

GW = 128


def _mesh():
    return plsc.VectorSubcoreMesh(core_axis_name="c", subcore_axis_name="s",
                                  num_cores=NC, num_subcores=NS)


def _scores_body(x_ref, w_ref, a_ref, s_ref, m_ref):
    i = pl.program_id(0)
    h = jnp.dot(x_ref[...], w_ref[...], preferred_element_type=jnp.float32)
    s = jnp.dot(h, a_ref[...], preferred_element_type=jnp.float32)
    s_ref[...] = s
    rows = i * BN + lax.broadcasted_iota(jnp.int32, s.shape, 0)
    sm = jnp.where(rows < N, s, -jnp.inf)
    bm = jnp.max(sm, axis=0, keepdims=True)

    @pl.when(i == 0)
    def _():
        m_ref[...] = bm

    @pl.when(i > 0)
    def _():
        m_ref[...] = jnp.maximum(m_ref[...], bm)


def _scores(x, W, A8):
    return pl.pallas_call(
        _scores_body,
        grid=(GN,),
        in_specs=[
            pl.BlockSpec((BN, F), lambda i: (i, 0)),
            pl.BlockSpec((F, F), lambda i: (0, 0)),
            pl.BlockSpec((F, 2 * H), lambda i: (0, 0)),
        ],
        out_specs=[
            pl.BlockSpec((BN, 2 * H), lambda i: (i, 0)),
            pl.BlockSpec((1, 2 * H), lambda i: (0, 0)),
        ],
        out_shape=[
            jax.ShapeDtypeStruct((N, 2 * H), jnp.float32),
            jax.ShapeDtypeStruct((1, 2 * H), jnp.float32),
        ],
    )(x, W, A8)


def _edge_body(src_ref, dst_ref, sa_ref, sb_ref, m_ref, ex_ref, dpart_ref,
               mv, isrc, idst, rsrc, rdst, exv, dnf, sem):
    cid = lax.axis_index("c")
    sid = lax.axis_index("s")
    wid = cid * NS + sid

    def _zr(r, c):
        dnf[pl.ds(r * L, L)] = jnp.zeros((L,), jnp.float32)
        return c
    lax.fori_loop(0, DNF // L, _zr, None)

    pltpu.sync_copy(m_ref, mv)

    iota = lax.iota(jnp.int32, L)
    pat = jnp.where(iota < H, 1.0, 0.0)
    mvec = mv[0, :]

    def _chunk(k, c):
        base = wid * EW + k * CH
        pltpu.sync_copy(src_ref.at[pl.ds(base, CH)], isrc)
        pltpu.sync_copy(dst_ref.at[pl.ds(base, CH)], idst)
        pltpu.async_copy(sa_ref.at[isrc], rsrc, sem).wait()
        pltpu.async_copy(sb_ref.at[idst], rdst, sem).wait()
        for e in range(CH):
            t = rsrc[e, pl.ds(0, L)] + rdst[e, pl.ds(0, L)]
            lr = jnp.where(t > 0, t, 0.2 * t)
            exv[e, :] = jnp.exp(lr - mvec) * pat
        pltpu.sync_copy(exv, ex_ref.at[pl.ds(base, CH)])

        @pl.when(base < E)
        def _():
            for v in range(CH // L):
                dl = idst[pl.ds(v * L, L)]
                for i in range(L):
                    e = v * L + i
                    plsc.addupdate(dnf.at[pl.ds(dl[i] * H, L)], exv[e, :])
        return c

    lax.fori_loop(0, NCH, _chunk, None)

    pltpu.sync_copy(dnf, dpart_ref.at[wid])


def _edge_pass(src_p, dst_p, sa, sb, mtab):
    fn = pl.kernel(
        _edge_body,
        out_type=[
            jax.ShapeDtypeStruct((EPAD, L), jnp.float32),
            jax.ShapeDtypeStruct((NW, DNF), jnp.float32),
        ],
        mesh=_mesh(),
        scratch_types=[
            pltpu.VMEM((1, L), jnp.float32),
            pltpu.VMEM((CH,), jnp.int32),
            pltpu.VMEM((CH,), jnp.int32),
            pltpu.VMEM((CH, GW), jnp.float32),
            pltpu.VMEM((CH, GW), jnp.float32),
            pltpu.VMEM((CH, L), jnp.float32),
            pltpu.VMEM((DNF,), jnp.float32),
            pltpu.SemaphoreType.DMA,
        ],
    )
    return fn(src_p, dst_p, sa, sb, mtab)


def _dsum_body(dpart_ref, dsum_ref, acc, tmp, tw):
    cid = lax.axis_index("c")
    sid = lax.axis_index("s")
    wid = cid * NS + sid
    W0 = DNF // NW
    R0 = DN // NW

    pltpu.sync_copy(dpart_ref.at[0, pl.ds(wid * W0, W0)], acc.at[pl.ds(0, W0)])

    def _zt(r, c):
        acc[pl.ds(W0 + r * L, L)] = jnp.zeros((L,), jnp.float32)
        return c
    lax.fori_loop(0, (acc.shape[0] - W0) // L, _zt, None)

    for w in range(1, NW):
        pltpu.sync_copy(dpart_ref.at[w, pl.ds(wid * W0, W0)], tmp)

        def _add(v, c):
            acc[pl.ds(v * L, L)] = acc[pl.ds(v * L, L)] + tmp[pl.ds(v * L, L)]
            return c
        lax.fori_loop(0, W0 // L, _add, None)

    iota = lax.iota(jnp.int32, L)
    pat = jnp.where(iota < H, 1.0, 0.0)

    def _zw(r, c):
        for j in range(GW // L):
            tw[r, pl.ds(j * L, L)] = jnp.zeros((L,), jnp.float32)
        return c
    lax.fori_loop(0, R0, _zw, None)

    def _w(r, c):
        tw[r, pl.ds(0, L)] = acc[pl.ds(r * H, L)] * pat
        return c
    lax.fori_loop(0, R0, _w, None)
    pltpu.sync_copy(tw, dsum_ref.at[pl.ds(wid * R0, R0)])


def _dsum(dpart):
    fn = pl.kernel(
        _dsum_body,
        out_type=jax.ShapeDtypeStruct((DN, GW), jnp.float32),
        mesh=_mesh(),
        scratch_types=[
            pltpu.VMEM((DNF // NW + L, ), jnp.float32),
            pltpu.VMEM((DNF // NW,), jnp.float32),
            pltpu.VMEM((DN // NW, GW), jnp.float32),
        ],
    )
    return fn(dpart)


def _dgather_body(dst_ref, dsum_ref, dg_ref, idst, db, g16, sem):
    cid = lax.axis_index("c")
    sid = lax.axis_index("s")
    wid = cid * NS + sid

    def _chunk(k, c):
        base = wid * EW + k * CH
        pltpu.sync_copy(dst_ref.at[pl.ds(base, CH)], idst)
        pltpu.async_copy(dsum_ref.at[idst], db, sem).wait()
        for e in range(CH):
            g16[e, :] = db[e, pl.ds(0, L)]
        pltpu.sync_copy(g16, dg_ref.at[pl.ds(base, CH)])
        return c

    lax.fori_loop(0, NCH, _chunk, None)


def _dgather(dst_p, dsum):
    fn = pl.kernel(
        _dgather_body,
        out_type=jax.ShapeDtypeStruct((EPAD, L), jnp.float32),
        mesh=_mesh(),
        scratch_types=[
            pltpu.VMEM((CH,), jnp.int32),
            pltpu.VMEM((CH, GW), jnp.float32),
            pltpu.VMEM((CH, L), jnp.float32),
            pltpu.SemaphoreType.DMA,
        ],
    )
    return fn(dst_p, dsum)


ABN = 2048


def _alpha_tc_body(ex_ref, dg_ref, a_ref):
    i = pl.program_id(0)
    acc = ex_ref[...] / (dg_ref[...] + 1e-16)
    s = jnp.sum(acc, axis=1, keepdims=True) * (1.0 / H)
    rows = i * ABN + lax.broadcasted_iota(jnp.int32, s.shape, 0)
    s = jnp.where(rows < E, s, 0.0)
    a_ref[...] = jnp.broadcast_to(s, (ABN, L))


def _alpha_tc(exT, dg):
    return pl.pallas_call(
        _alpha_tc_body,
        grid=(EPAD // ABN,),
        in_specs=[
            pl.BlockSpec((ABN, L), lambda i: (i, 0)),
            pl.BlockSpec((ABN, L), lambda i: (i, 0)),
        ],
        out_specs=pl.BlockSpec((ABN, L), lambda i: (i, 0)),
        out_shape=jax.ShapeDtypeStruct((EPAD, L), jnp.float32),
    )(exT, dg)


F2 = F // 2
FJ = F2 // L
UR = 40
NU = NHALF // UR


def _dif_body(z0_ref, z1_ref, src_ref, alpha_ref, dst_ref,
              zo0_ref, zo1_ref,
              zb2, isrc, idst, sidx, ab, rows, uagg, uz, agg_sh, sem):
    cid = lax.axis_index("c")
    sid = lax.axis_index("s")
    zin = [z0_ref, z1_ref]
    zout = [zo0_ref, zo1_ref]

    for r in range(DC):
        for j in range(FJ):
            zb2[r, pl.ds(j * L, L)] = jnp.zeros((L,), jnp.float32)

    for half in range(2):
        for cix in range(SROWS // NS // DC):
            pltpu.sync_copy(
                zb2, agg_sh.at[pl.ds(sid * (SROWS // NS) + cix * DC, DC)])
        plsc.subcore_barrier()

        def _chunk(k, c):
            base = sid * ETILE + k * DC
            pltpu.sync_copy(src_ref.at[pl.ds(base, DC)], isrc)
            pltpu.sync_copy(dst_ref.at[pl.ds(base, DC)], idst)
            pltpu.sync_copy(alpha_ref.at[pl.ds(base, DC)], ab)
            pltpu.async_copy(zin[half].at[isrc], rows, sem).wait()
            for e in range(DC):
                asp = ab[e, :]
                for j in range(FJ):
                    rows[e, pl.ds(j * L, L)] = rows[e, pl.ds(j * L, L)] * asp
            for v in range(DC // L):
                dl = idst[pl.ds(v * L, L)]
                loc = dl - cid * NHALF
                ok = (loc >= 0) & (loc < NHALF)
                sidx[pl.ds(v * L, L)] = jnp.where(ok, loc, DUMMY)
            pltpu.sync_copy(rows, agg_sh.at[sidx], add=True)
            return c

        lax.fori_loop(0, NDC, _chunk, None)
        plsc.subcore_barrier()

        def _upd(u, c):
            idx = sid * 8 + u

            @pl.when(idx < NU)
            def _():
                g = cid * NHALF + idx * UR
                pltpu.sync_copy(agg_sh.at[pl.ds(idx * UR, UR)], uagg)
                pltpu.sync_copy(zin[half].at[pl.ds(g, UR)], uz)

                def _row(r, c2):
                    for j in range(FJ):
                        uz[r, pl.ds(j * L, L)] = (
                            uz[r, pl.ds(j * L, L)] * (1.0 - DT)
                            + uagg[r, pl.ds(j * L, L)] * DT)
                    return c2
                lax.fori_loop(0, UR, _row, None)
                pltpu.sync_copy(uz, zout[half].at[pl.ds(g, UR)])
            return c

        lax.fori_loop(0, 8, _upd, None)
        plsc.subcore_barrier()


def _dif_step(z0, z1, src_p, alpha, dst_p):
    fn = pl.kernel(
        _dif_body,
        out_type=[
            jax.ShapeDtypeStruct((N, F2), jnp.float32),
            jax.ShapeDtypeStruct((N, F2), jnp.float32),
        ],
        mesh=_mesh(),
        scratch_types=[
            pltpu.VMEM((DC, F2), jnp.float32),
            pltpu.VMEM((DC,), jnp.int32),
            pltpu.VMEM((DC,), jnp.int32),
            pltpu.VMEM((DC,), jnp.int32),
            pltpu.VMEM((DC, L), jnp.float32),
            pltpu.VMEM((DC, F2), jnp.float32),
            pltpu.VMEM((UR, F2), jnp.float32),
            pltpu.VMEM((UR, F2), jnp.float32),
            pltpu.VMEM_SHARED((SROWS, F2), jnp.float32),
            pltpu.SemaphoreType.DMA,
        ],
    )
    return tuple(fn(z0, z1, src_p, alpha, dst_p))


def _elu_body(z_ref, o_ref):
    z = z_ref[...]
    o_ref[...] = jnp.where(z > 0, z, jnp.exp(z) - 1.0)


def _elu(z):
    return pl.pallas_call(
        _elu_body,
        grid=(GN,),
        in_specs=[pl.BlockSpec((BN, F), lambda i: (i, 0))],
        out_specs=pl.BlockSpec((BN, F), lambda i: (i, 0)),
        out_shape=jax.ShapeDtypeStruct((N, F), jnp.float32),
    )(z)


def _decoder_body(fi_ref, fj_ref, o_ref):
    o_ref[...] = lax.dot_general(
        fi_ref[...], fj_ref[...],
        (((1,), (1,)), ((), ())),
        preferred_element_type=jnp.float32)


def _decoder(f):
    return pl.pallas_call(
        _decoder_body,
        grid=(GN, GN),
        in_specs=[
            pl.BlockSpec((BN, F), lambda i, j: (i, 0)),
            pl.BlockSpec((BN, F), lambda i, j: (j, 0)),
        ],
        out_specs=pl.BlockSpec((BN, BN), lambda i, j: (i, j)),
        out_shape=jax.ShapeDtypeStruct((N, N), jnp.float32),
    )(f, f)


def kernel(x, edge_index, W, a_src, a_dst):
    src = edge_index[0]
    dst = edge_index[1]

    eye = jnp.eye(H, dtype=jnp.float32)
    Bs = (eye[:, None, :] * a_src[:, :, None]).reshape(F, H)
    Bd = (eye[:, None, :] * a_dst[:, :, None]).reshape(F, H)
    A8 = jnp.concatenate([Bs, Bd], axis=1)

    s8, smax = _scores(x, W, A8)
    m4 = smax[0, :H] + smax[0, H:]
    m4 = jnp.where(m4 > 0, m4, 0.2 * m4)
    mtab = jnp.pad(m4, (0, L - H))[None, :]

    sa = jnp.pad(s8[:, :H], ((0, 0), (0, GW - H)))
    sb = jnp.pad(s8[:, H:], ((0, 0), (0, GW - H)))

    pad = jnp.zeros((EPAD - E,), jnp.int32)
    src_p = jnp.concatenate([src, pad])
    dst_p = jnp.concatenate([dst, pad])

    exT, dpart = _edge_pass(src_p, dst_p, sa, sb, mtab)
    dsum = _dsum(dpart)
    dg = _dgather(dst_p, dsum)
    alpha = _alpha_tc(exT, dg)
    def _step(_, zz):
        return _dif_step(zz[0], zz[1], src_p, alpha, dst_p)

    z0, z1 = lax.fori_loop(0, STEPS, _step, (x[:, :F2], x[:, F2:]))
    z = jnp.concatenate([z0, z1], axis=1)

    f = _elu(z)
    recon = _decoder(f)
    return (f, recon, x, z)


# speedup vs baseline: 1.2133x; 1.2133x over previous
import jax
import jax.numpy as jnp
from jax import lax
from jax.experimental import pallas as pl
from jax.experimental.pallas import tpu as pltpu
from jax.experimental.pallas import tpu_sc as plsc

N = 10000
F = 256
H = 4
E = 160000
STEPS = 4
DT = 0.25

NC = 2
NS = 16
NW = NC * NS
L = 16

EPAD = 163840
EW = EPAD // NW
CH = 128
NCH = EW // CH

ETILE = EPAD // NS
DC = 128
NDC = ETILE // DC

NHALF = N // NC
SROWS = 5120
DUMMY = NHALF

DN = 10240
DTILE = DN // NS
DNF = DN * H

BN = 512
GN = (N + BN - 1) // BN

GW = 128


def _mesh():
    return plsc.VectorSubcoreMesh(core_axis_name="c", subcore_axis_name="s",
                                  num_cores=NC, num_subcores=NS)


def _scores_body(x_ref, w_ref, a_ref, s_ref, m_ref):
    i = pl.program_id(0)
    h = jnp.dot(x_ref[...], w_ref[...], preferred_element_type=jnp.float32)
    s = jnp.dot(h, a_ref[...], preferred_element_type=jnp.float32)
    s_ref[...] = s
    rows = i * BN + lax.broadcasted_iota(jnp.int32, s.shape, 0)
    sm = jnp.where(rows < N, s, -jnp.inf)
    bm = jnp.max(sm, axis=0, keepdims=True)

    @pl.when(i == 0)
    def _():
        m_ref[...] = bm

    @pl.when(i > 0)
    def _():
        m_ref[...] = jnp.maximum(m_ref[...], bm)


def _scores(x, W, A8):
    return pl.pallas_call(
        _scores_body,
        grid=(GN,),
        in_specs=[
            pl.BlockSpec((BN, F), lambda i: (i, 0)),
            pl.BlockSpec((F, F), lambda i: (0, 0)),
            pl.BlockSpec((F, 2 * H), lambda i: (0, 0)),
        ],
        out_specs=[
            pl.BlockSpec((BN, 2 * H), lambda i: (i, 0)),
            pl.BlockSpec((1, 2 * H), lambda i: (0, 0)),
        ],
        out_shape=[
            jax.ShapeDtypeStruct((N, 2 * H), jnp.float32),
            jax.ShapeDtypeStruct((1, 2 * H), jnp.float32),
        ],
    )(x, W, A8)


def _edge_body(src_ref, dst_ref, sa_ref, sb_ref, m_ref, ex_ref, dpart_ref,
               mv, isrc, idst, rsrc, rdst, exv, dnf, sem):
    cid = lax.axis_index("c")
    sid = lax.axis_index("s")
    wid = cid * NS + sid

    def _zr(r, c):
        dnf[pl.ds(r * L, L)] = jnp.zeros((L,), jnp.float32)
        return c
    lax.fori_loop(0, DNF // L, _zr, None)

    pltpu.sync_copy(m_ref, mv)

    iota = lax.iota(jnp.int32, L)
    pat = jnp.where(iota < H, 1.0, 0.0)
    mvec = mv[0, :]

    def _chunk(k, c):
        base = wid * EW + k * CH
        pltpu.sync_copy(src_ref.at[pl.ds(base, CH)], isrc)
        pltpu.sync_copy(dst_ref.at[pl.ds(base, CH)], idst)
        pltpu.async_copy(sa_ref.at[isrc], rsrc, sem).wait()
        pltpu.async_copy(sb_ref.at[idst], rdst, sem).wait()
        for e in range(CH):
            t = rsrc[e, pl.ds(0, L)] + rdst[e, pl.ds(0, L)]
            lr = jnp.where(t > 0, t, 0.2 * t)
            exv[e, :] = jnp.exp(lr - mvec) * pat
        pltpu.sync_copy(exv, ex_ref.at[pl.ds(base, CH)])

        @pl.when(base < E)
        def _():
            for v in range(CH // L):
                dl = idst[pl.ds(v * L, L)]
                for i in range(L):
                    e = v * L + i
                    plsc.addupdate(dnf.at[pl.ds(dl[i] * H, L)], exv[e, :])
        return c

    lax.fori_loop(0, NCH, _chunk, None)

    pltpu.sync_copy(dnf, dpart_ref.at[wid])


def _edge_pass(src_p, dst_p, sa, sb, mtab):
    fn = pl.kernel(
        _edge_body,
        out_type=[
            jax.ShapeDtypeStruct((EPAD, L), jnp.float32),
            jax.ShapeDtypeStruct((NW, DNF), jnp.float32),
        ],
        mesh=_mesh(),
        scratch_types=[
            pltpu.VMEM((1, L), jnp.float32),
            pltpu.VMEM((CH,), jnp.int32),
            pltpu.VMEM((CH,), jnp.int32),
            pltpu.VMEM((CH, GW), jnp.float32),
            pltpu.VMEM((CH, GW), jnp.float32),
            pltpu.VMEM((CH, L), jnp.float32),
            pltpu.VMEM((DNF,), jnp.float32),
            pltpu.SemaphoreType.DMA,
        ],
    )
    return fn(src_p, dst_p, sa, sb, mtab)


def _dsum_body(dpart_ref, dsum_ref, acc, tmp, tw):
    cid = lax.axis_index("c")
    sid = lax.axis_index("s")
    wid = cid * NS + sid
    W0 = DNF // NW
    R0 = DN // NW

    pltpu.sync_copy(dpart_ref.at[0, pl.ds(wid * W0, W0)], acc.at[pl.ds(0, W0)])

    def _zt(r, c):
        acc[pl.ds(W0 + r * L, L)] = jnp.zeros((L,), jnp.float32)
        return c
    lax.fori_loop(0, (acc.shape[0] - W0) // L, _zt, None)

    for w in range(1, NW):
        pltpu.sync_copy(dpart_ref.at[w, pl.ds(wid * W0, W0)], tmp)

        def _add(v, c):
            acc[pl.ds(v * L, L)] = acc[pl.ds(v * L, L)] + tmp[pl.ds(v * L, L)]
            return c
        lax.fori_loop(0, W0 // L, _add, None)

    iota = lax.iota(jnp.int32, L)
    pat = jnp.where(iota < H, 1.0, 0.0)

    def _zw(r, c):
        for j in range(GW // L):
            tw[r, pl.ds(j * L, L)] = jnp.zeros((L,), jnp.float32)
        return c
    lax.fori_loop(0, R0, _zw, None)

    def _w(r, c):
        tw[r, pl.ds(0, L)] = acc[pl.ds(r * H, L)] * pat
        return c
    lax.fori_loop(0, R0, _w, None)
    pltpu.sync_copy(tw, dsum_ref.at[pl.ds(wid * R0, R0)])


def _dsum(dpart):
    fn = pl.kernel(
        _dsum_body,
        out_type=jax.ShapeDtypeStruct((DN, GW), jnp.float32),
        mesh=_mesh(),
        scratch_types=[
            pltpu.VMEM((DNF // NW + L, ), jnp.float32),
            pltpu.VMEM((DNF // NW,), jnp.float32),
            pltpu.VMEM((DN // NW, GW), jnp.float32),
        ],
    )
    return fn(dpart)


def _dgather_body(dst_ref, dsum_ref, dg_ref, idst, db, g16, sem):
    cid = lax.axis_index("c")
    sid = lax.axis_index("s")
    wid = cid * NS + sid

    def _chunk(k, c):
        base = wid * EW + k * CH
        pltpu.sync_copy(dst_ref.at[pl.ds(base, CH)], idst)
        pltpu.async_copy(dsum_ref.at[idst], db, sem).wait()
        for e in range(CH):
            g16[e, :] = db[e, pl.ds(0, L)]
        pltpu.sync_copy(g16, dg_ref.at[pl.ds(base, CH)])
        return c

    lax.fori_loop(0, NCH, _chunk, None)


def _dgather(dst_p, dsum):
    fn = pl.kernel(
        _dgather_body,
        out_type=jax.ShapeDtypeStruct((EPAD, L), jnp.float32),
        mesh=_mesh(),
        scratch_types=[
            pltpu.VMEM((CH,), jnp.int32),
            pltpu.VMEM((CH, GW), jnp.float32),
            pltpu.VMEM((CH, L), jnp.float32),
            pltpu.SemaphoreType.DMA,
        ],
    )
    return fn(dst_p, dsum)


ABN = 2048


def _alpha_tc_body(ex_ref, dg_ref, a_ref):
    i = pl.program_id(0)
    acc = ex_ref[...] / (dg_ref[...] + 1e-16)
    s = jnp.sum(acc, axis=1, keepdims=True) * (1.0 / H)
    rows = i * ABN + lax.broadcasted_iota(jnp.int32, s.shape, 0)
    s = jnp.where(rows < E, s, 0.0)
    a_ref[...] = jnp.broadcast_to(s, (ABN, L))


def _alpha_tc(exT, dg):
    return pl.pallas_call(
        _alpha_tc_body,
        grid=(EPAD // ABN,),
        in_specs=[
            pl.BlockSpec((ABN, L), lambda i: (i, 0)),
            pl.BlockSpec((ABN, L), lambda i: (i, 0)),
        ],
        out_specs=pl.BlockSpec((ABN, L), lambda i: (i, 0)),
        out_shape=jax.ShapeDtypeStruct((EPAD, L), jnp.float32),
    )(exT, dg)


F2 = F // 2
FJ = F2 // L
UR = 40
NU = NHALF // UR


def _dif_body(z0_ref, z1_ref, src_ref, alpha_ref, dst_ref,
              zo0_ref, zo1_ref,
              zb2, isrc_a, isrc_b, idst_a, idst_b, sidx,
              ab_a, ab_b, rows_a, rows_b, uagg, uz, agg_sh, sem_a, sem_b):
    cid = lax.axis_index("c")
    sid = lax.axis_index("s")
    zin = [z0_ref, z1_ref]
    zout = [zo0_ref, zo1_ref]
    bufs = ((isrc_a, idst_a, ab_a, rows_a, sem_a),
            (isrc_b, idst_b, ab_b, rows_b, sem_b))

    for r in range(64):
        for j in range(FJ):
            zb2[r, pl.ds(j * L, L)] = jnp.zeros((L,), jnp.float32)

    def _fetch(zr, k, bset):
        isrc, idst, ab, rows, sem = bset
        base = sid * ETILE + k * DC
        pltpu.sync_copy(src_ref.at[pl.ds(base, DC)], isrc)
        pltpu.sync_copy(dst_ref.at[pl.ds(base, DC)], idst)
        pltpu.sync_copy(alpha_ref.at[pl.ds(base, DC)], ab)
        pltpu.async_copy(zr.at[isrc], rows, sem)

    def _consume(zr, k, bset):
        isrc, idst, ab, rows, sem = bset
        pltpu.make_async_copy(zr.at[isrc], rows, sem).wait()
        for e in range(DC):
            asp = ab[e, :]
            for j in range(FJ):
                rows[e, pl.ds(j * L, L)] = rows[e, pl.ds(j * L, L)] * asp
        for v in range(DC // L):
            dl = idst[pl.ds(v * L, L)]
            loc = dl - cid * NHALF
            ok = (loc >= 0) & (loc < NHALF)
            sidx[pl.ds(v * L, L)] = jnp.where(ok, loc, DUMMY)
        pltpu.sync_copy(rows, agg_sh.at[sidx], add=True)

    for half in range(2):
        for cix in range(SROWS // NS // 64):
            pltpu.sync_copy(
                zb2, agg_sh.at[pl.ds(sid * (SROWS // NS) + cix * 64, 64)])
        plsc.subcore_barrier()

        zr = zin[half]
        _fetch(zr, 0, bufs[0])

        def _pair(k2, c):
            for b in range(2):
                k = k2 * 2 + b

                @pl.when(k + 1 < NDC)
                def _():
                    _fetch(zr, k + 1, bufs[1 - b])
                _consume(zr, k, bufs[b])
            return c

        lax.fori_loop(0, NDC // 2, _pair, None)
        plsc.subcore_barrier()

        def _upd(u, c):
            idx = sid * 8 + u

            @pl.when(idx < NU)
            def _():
                g = cid * NHALF + idx * UR
                pltpu.sync_copy(agg_sh.at[pl.ds(idx * UR, UR)], uagg)
                pltpu.sync_copy(zin[half].at[pl.ds(g, UR)], uz)

                def _row(r, c2):
                    for j in range(FJ):
                        uz[r, pl.ds(j * L, L)] = (
                            uz[r, pl.ds(j * L, L)] * (1.0 - DT)
                            + uagg[r, pl.ds(j * L, L)] * DT)
                    return c2
                lax.fori_loop(0, UR, _row, None)
                pltpu.sync_copy(uz, zout[half].at[pl.ds(g, UR)])
            return c

        lax.fori_loop(0, 8, _upd, None)
        plsc.subcore_barrier()


def _dif_step(z0, z1, src_p, alpha, dst_p):
    fn = pl.kernel(
        _dif_body,
        out_type=[
            jax.ShapeDtypeStruct((N, F2), jnp.float32),
            jax.ShapeDtypeStruct((N, F2), jnp.float32),
        ],
        mesh=_mesh(),
        scratch_types=[
            pltpu.VMEM((64, F2), jnp.float32),
            pltpu.VMEM((DC,), jnp.int32),
            pltpu.VMEM((DC,), jnp.int32),
            pltpu.VMEM((DC,), jnp.int32),
            pltpu.VMEM((DC,), jnp.int32),
            pltpu.VMEM((DC,), jnp.int32),
            pltpu.VMEM((DC, L), jnp.float32),
            pltpu.VMEM((DC, L), jnp.float32),
            pltpu.VMEM((DC, F2), jnp.float32),
            pltpu.VMEM((DC, F2), jnp.float32),
            pltpu.VMEM((UR, F2), jnp.float32),
            pltpu.VMEM((UR, F2), jnp.float32),
            pltpu.VMEM_SHARED((SROWS, F2), jnp.float32),
            pltpu.SemaphoreType.DMA,
            pltpu.SemaphoreType.DMA,
        ],
    )
    return tuple(fn(z0, z1, src_p, alpha, dst_p))


def _elu_body(z_ref, o_ref):
    z = z_ref[...]
    o_ref[...] = jnp.where(z > 0, z, jnp.exp(z) - 1.0)


def _elu(z):
    return pl.pallas_call(
        _elu_body,
        grid=(GN,),
        in_specs=[pl.BlockSpec((BN, F), lambda i: (i, 0))],
        out_specs=pl.BlockSpec((BN, F), lambda i: (i, 0)),
        out_shape=jax.ShapeDtypeStruct((N, F), jnp.float32),
    )(z)


def _decoder_body(fi_ref, fj_ref, o_ref):
    o_ref[...] = lax.dot_general(
        fi_ref[...], fj_ref[...],
        (((1,), (1,)), ((), ())),
        preferred_element_type=jnp.float32)


def _decoder(f):
    return pl.pallas_call(
        _decoder_body,
        grid=(GN, GN),
        in_specs=[
            pl.BlockSpec((BN, F), lambda i, j: (i, 0)),
            pl.BlockSpec((BN, F), lambda i, j: (j, 0)),
        ],
        out_specs=pl.BlockSpec((BN, BN), lambda i, j: (i, j)),
        out_shape=jax.ShapeDtypeStruct((N, N), jnp.float32),
    )(f, f)


def kernel(x, edge_index, W, a_src, a_dst):
    src = edge_index[0]
    dst = edge_index[1]

    eye = jnp.eye(H, dtype=jnp.float32)
    Bs = (eye[:, None, :] * a_src[:, :, None]).reshape(F, H)
    Bd = (eye[:, None, :] * a_dst[:, :, None]).reshape(F, H)
    A8 = jnp.concatenate([Bs, Bd], axis=1)

    s8, smax = _scores(x, W, A8)
    m4 = smax[0, :H] + smax[0, H:]
    m4 = jnp.where(m4 > 0, m4, 0.2 * m4)
    mtab = jnp.pad(m4, (0, L - H))[None, :]

    sa = jnp.pad(s8[:, :H], ((0, 0), (0, GW - H)))
    sb = jnp.pad(s8[:, H:], ((0, 0), (0, GW - H)))

    pad = jnp.zeros((EPAD - E,), jnp.int32)
    src_p = jnp.concatenate([src, pad])
    dst_p = jnp.concatenate([dst, pad])

    exT, dpart = _edge_pass(src_p, dst_p, sa, sb, mtab)
    dsum = _dsum(dpart)
    dg = _dgather(dst_p, dsum)
    alpha = _alpha_tc(exT, dg)
    def _step(_, zz):
        return _dif_step(zz[0], zz[1], src_p, alpha, dst_p)

    z0, z1 = lax.fori_loop(0, STEPS, _step, (x[:, :F2], x[:, F2:]))
    z = jnp.concatenate([z0, z1], axis=1)

    f = _elu(z)
    recon = _decoder(f)
    return (f, recon, x, z)


# speedup vs baseline: 1.2791x; 1.0543x over previous
import jax
import jax.numpy as jnp
from jax import lax
from jax.experimental import pallas as pl
from jax.experimental.pallas import tpu as pltpu
from jax.experimental.pallas import tpu_sc as plsc

N = 10000
F = 256
H = 4
E = 160000
STEPS = 4
DT = 0.25

NC = 2
NS = 16
NW = NC * NS
L = 16

EPAD = 163840
EW = EPAD // NW
CH = 128
NCH = EW // CH

ETILE = EPAD // NS
DC = 128
NDC = ETILE // DC

NHALF = N // NC
SROWS = 5120
DUMMY = NHALF

DN = 10240
DTILE = DN // NS
DNF = DN * H

BN = 512
GN = (N + BN - 1) // BN

GW = 128


def _mesh():
    return plsc.VectorSubcoreMesh(core_axis_name="c", subcore_axis_name="s",
                                  num_cores=NC, num_subcores=NS)


def _scores_body(x_ref, w_ref, a_ref, s_ref, m_ref):
    i = pl.program_id(0)
    h = jnp.dot(x_ref[...], w_ref[...], preferred_element_type=jnp.float32)
    s = jnp.dot(h, a_ref[...], preferred_element_type=jnp.float32)
    s_ref[...] = s
    rows = i * BN + lax.broadcasted_iota(jnp.int32, s.shape, 0)
    sm = jnp.where(rows < N, s, -jnp.inf)
    bm = jnp.max(sm, axis=0, keepdims=True)

    @pl.when(i == 0)
    def _():
        m_ref[...] = bm

    @pl.when(i > 0)
    def _():
        m_ref[...] = jnp.maximum(m_ref[...], bm)


def _scores(x, W, A8):
    return pl.pallas_call(
        _scores_body,
        grid=(GN,),
        in_specs=[
            pl.BlockSpec((BN, F), lambda i: (i, 0)),
            pl.BlockSpec((F, F), lambda i: (0, 0)),
            pl.BlockSpec((F, 2 * H), lambda i: (0, 0)),
        ],
        out_specs=[
            pl.BlockSpec((BN, 2 * H), lambda i: (i, 0)),
            pl.BlockSpec((1, 2 * H), lambda i: (0, 0)),
        ],
        out_shape=[
            jax.ShapeDtypeStruct((N, 2 * H), jnp.float32),
            jax.ShapeDtypeStruct((1, 2 * H), jnp.float32),
        ],
    )(x, W, A8)


def _edge_body(src_ref, dst_ref, sa_ref, sb_ref, m_ref, ex_ref, dpart_ref,
               mv, isrc, idst, rsrc, rdst, exv, dnf, sem):
    cid = lax.axis_index("c")
    sid = lax.axis_index("s")
    wid = cid * NS + sid

    def _zr(r, c):
        dnf[pl.ds(r * L, L)] = jnp.zeros((L,), jnp.float32)
        return c
    lax.fori_loop(0, DNF // L, _zr, None)

    pltpu.sync_copy(m_ref, mv)

    iota = lax.iota(jnp.int32, L)
    pat = jnp.where(iota < H, 1.0, 0.0)
    mvec = mv[0, :]

    def _chunk(k, c):
        base = wid * EW + k * CH
        pltpu.sync_copy(src_ref.at[pl.ds(base, CH)], isrc)
        pltpu.sync_copy(dst_ref.at[pl.ds(base, CH)], idst)
        pltpu.async_copy(sa_ref.at[isrc], rsrc, sem).wait()
        pltpu.async_copy(sb_ref.at[idst], rdst, sem).wait()
        for e in range(CH):
            t = rsrc[e, pl.ds(0, L)] + rdst[e, pl.ds(0, L)]
            lr = jnp.where(t > 0, t, 0.2 * t)
            exv[e, :] = jnp.exp(lr - mvec) * pat
        pltpu.sync_copy(exv, ex_ref.at[pl.ds(base, CH)])

        @pl.when(base < E)
        def _():
            for v in range(CH // L):
                dl = idst[pl.ds(v * L, L)]
                for i in range(L):
                    e = v * L + i
                    plsc.addupdate(dnf.at[pl.ds(dl[i] * H, L)], exv[e, :])
        return c

    lax.fori_loop(0, NCH, _chunk, None)

    pltpu.sync_copy(dnf, dpart_ref.at[wid, pl.ds(0, DNF)])


def _edge_pass(src_p, dst_p, sa, sb, mtab):
    fn = pl.kernel(
        _edge_body,
        out_type=[
            jax.ShapeDtypeStruct((EPAD, L), jnp.float32),
            jax.ShapeDtypeStruct((NW, 2 * DNF), jnp.float32),
        ],
        mesh=_mesh(),
        scratch_types=[
            pltpu.VMEM((1, L), jnp.float32),
            pltpu.VMEM((CH,), jnp.int32),
            pltpu.VMEM((CH,), jnp.int32),
            pltpu.VMEM((CH, GW), jnp.float32),
            pltpu.VMEM((CH, GW), jnp.float32),
            pltpu.VMEM((CH, L), jnp.float32),
            pltpu.VMEM((DNF,), jnp.float32),
            pltpu.SemaphoreType.DMA,
        ],
    )
    return fn(src_p, dst_p, sa, sb, mtab)


def _dsum_body(dpart_ref, dsum_ref, acc, tmp, tw):
    cid = lax.axis_index("c")
    sid = lax.axis_index("s")
    wid = cid * NS + sid
    W0 = DNF // NW
    R0 = DN // NW

    pltpu.sync_copy(dpart_ref.at[0, pl.ds(wid * W0, W0)], acc.at[pl.ds(0, W0)])

    def _zt(r, c):
        acc[pl.ds(W0 + r * L, L)] = jnp.zeros((L,), jnp.float32)
        return c
    lax.fori_loop(0, (acc.shape[0] - W0) // L, _zt, None)

    for w in range(1, NW):
        pltpu.sync_copy(dpart_ref.at[w, pl.ds(wid * W0, W0)], tmp)

        def _add(v, c):
            acc[pl.ds(v * L, L)] = acc[pl.ds(v * L, L)] + tmp[pl.ds(v * L, L)]
            return c
        lax.fori_loop(0, W0 // L, _add, None)

    G0 = R0 // 4

    def _zw(r, c):
        for j in range(GW // L):
            tw[r, pl.ds(j * L, L)] = jnp.zeros((L,), jnp.float32)
        return c
    lax.fori_loop(0, G0, _zw, None)

    def _w(r, c):
        tw[r, pl.ds(0, L)] = acc[pl.ds(r * L, L)]
        return c
    lax.fori_loop(0, G0, _w, None)
    pltpu.sync_copy(tw, dsum_ref.at[pl.ds(wid * G0, G0)])


def _dsum(dpart):
    fn = pl.kernel(
        _dsum_body,
        out_type=jax.ShapeDtypeStruct((DN // 4, GW), jnp.float32),
        mesh=_mesh(),
        scratch_types=[
            pltpu.VMEM((DNF // NW + L, ), jnp.float32),
            pltpu.VMEM((DNF // NW,), jnp.float32),
            pltpu.VMEM((DN // NW // 4, GW), jnp.float32),
        ],
    )
    return fn(dpart)


def _dgather_body(dst_ref, dsum_ref, dg_ref, idst, gix, db, g16, sem):
    cid = lax.axis_index("c")
    sid = lax.axis_index("s")
    wid = cid * NS + sid

    def _chunk(k, c):
        base = wid * EW + k * CH
        pltpu.sync_copy(dst_ref.at[pl.ds(base, CH)], idst)
        for v in range(CH // L):
            gix[pl.ds(v * L, L)] = lax.shift_right_logical(
                idst[pl.ds(v * L, L)], 2)
        pltpu.async_copy(dsum_ref.at[gix], db, sem).wait()
        for e in range(CH):
            g16[e, :] = db[e, pl.ds(0, L)]
        pltpu.sync_copy(g16, dg_ref.at[pl.ds(base, CH)])
        return c

    lax.fori_loop(0, NCH, _chunk, None)


def _dgather(dst_p, dsum):
    fn = pl.kernel(
        _dgather_body,
        out_type=jax.ShapeDtypeStruct((EPAD, L), jnp.float32),
        mesh=_mesh(),
        scratch_types=[
            pltpu.VMEM((CH,), jnp.int32),
            pltpu.VMEM((CH,), jnp.int32),
            pltpu.VMEM((CH, GW), jnp.float32),
            pltpu.VMEM((CH, L), jnp.float32),
            pltpu.SemaphoreType.DMA,
        ],
    )
    return fn(dst_p, dsum)


ABN = 2048


def _alpha_tc_body(ex_ref, dg_ref, dm_ref, a_ref):
    i = pl.program_id(0)
    m = dm_ref[...]
    d4 = jnp.zeros((ABN, H), jnp.float32)
    for k in range(4):
        d4 = jnp.where(m == k, dg_ref[:, 4 * k:4 * k + 4], d4)
    acc = ex_ref[:, :H] / (d4 + 1e-16)
    s = jnp.sum(acc, axis=1, keepdims=True) * (1.0 / H)
    rows = i * ABN + lax.broadcasted_iota(jnp.int32, s.shape, 0)
    s = jnp.where(rows < E, s, 0.0)
    a_ref[...] = jnp.broadcast_to(s, (ABN, L))


def _alpha_tc(exT, dg, dm):
    return pl.pallas_call(
        _alpha_tc_body,
        grid=(EPAD // ABN,),
        in_specs=[
            pl.BlockSpec((ABN, L), lambda i: (i, 0)),
            pl.BlockSpec((ABN, L), lambda i: (i, 0)),
            pl.BlockSpec((ABN, 1), lambda i: (i, 0)),
        ],
        out_specs=pl.BlockSpec((ABN, L), lambda i: (i, 0)),
        out_shape=jax.ShapeDtypeStruct((EPAD, L), jnp.float32),
    )(exT, dg, dm)


F2 = F // 2
FJ = F2 // L
UR = 40
NU = NHALF // UR


def _dif_body(z0_ref, z1_ref, src_ref, alpha_ref, dst_ref,
              zo0_ref, zo1_ref,
              zb2, isrc_a, isrc_b, idst_a, idst_b, sidx,
              ab_a, ab_b, rows_a, rows_b, uagg, uz, agg_sh, sem_a, sem_b):
    cid = lax.axis_index("c")
    sid = lax.axis_index("s")
    zin = [z0_ref, z1_ref]
    zout = [zo0_ref, zo1_ref]
    bufs = ((isrc_a, idst_a, ab_a, rows_a, sem_a),
            (isrc_b, idst_b, ab_b, rows_b, sem_b))

    for r in range(UR):
        for j in range(FJ):
            zb2[r, pl.ds(j * L, L)] = jnp.zeros((L,), jnp.float32)

    def _fetch(zr, k, bset):
        isrc, idst, ab, rows, sem = bset
        base = sid * ETILE + k * DC
        pltpu.async_copy(src_ref.at[pl.ds(base, DC)], isrc, sem)

    def _fetch2(zr, k, bset):
        isrc, idst, ab, rows, sem = bset
        base = sid * ETILE + k * DC
        pltpu.make_async_copy(src_ref.at[pl.ds(0, DC)], isrc, sem).wait()
        pltpu.async_copy(zr.at[isrc], rows, sem)
        pltpu.async_copy(dst_ref.at[pl.ds(base, DC)], idst, sem)
        pltpu.async_copy(alpha_ref.at[pl.ds(base, DC)], ab, sem)

    def _consume(zr, k, bset):
        isrc, idst, ab, rows, sem = bset
        pltpu.make_async_copy(dst_ref.at[pl.ds(0, DC)], idst, sem).wait()
        pltpu.make_async_copy(alpha_ref.at[pl.ds(0, DC)], ab, sem).wait()
        pltpu.make_async_copy(zr.at[isrc], rows, sem).wait()
        for e in range(DC):
            asp = ab[e, :]
            for j in range(FJ):
                rows[e, pl.ds(j * L, L)] = rows[e, pl.ds(j * L, L)] * asp
        for v in range(DC // L):
            dl = idst[pl.ds(v * L, L)]
            loc = dl - cid * NHALF
            ok = (loc >= 0) & (loc < NHALF)
            sidx[pl.ds(v * L, L)] = jnp.where(ok, loc, DUMMY)
        pltpu.sync_copy(rows, agg_sh.at[sidx], add=True)

    for half in range(2):
        for u in range(SROWS // NS // UR):
            pltpu.sync_copy(
                zb2, agg_sh.at[pl.ds((sid * (SROWS // NS // UR) + u) * UR,
                                     UR)])
        plsc.subcore_barrier()

        zr = zin[half]
        _fetch(zr, 0, bufs[0])
        _fetch2(zr, 0, bufs[0])

        def _pair(k2, c):
            for b in range(2):
                k = k2 * 2 + b

                @pl.when(k + 1 < NDC)
                def _():
                    _fetch(zr, k + 1, bufs[1 - b])
                    _fetch2(zr, k + 1, bufs[1 - b])
                _consume(zr, k, bufs[b])
            return c

        lax.fori_loop(0, NDC // 2, _pair, None)
        plsc.subcore_barrier()

        def _upd(u, c):
            idx = sid * (NU // NS + 1) + u

            @pl.when(idx < NU)
            def _():
                g = cid * NHALF + idx * UR
                pltpu.sync_copy(agg_sh.at[pl.ds(idx * UR, UR)], uagg)
                pltpu.sync_copy(zr.at[pl.ds(g, UR)], uz)

                def _row(r, c2):
                    for j in range(FJ):
                        uz[r, pl.ds(j * L, L)] = (
                            uz[r, pl.ds(j * L, L)] * (1.0 - DT)
                            + uagg[r, pl.ds(j * L, L)] * DT)
                    return c2
                lax.fori_loop(0, UR, _row, None)
                pltpu.sync_copy(uz, zout[half].at[pl.ds(g, UR)])
            return c

        lax.fori_loop(0, NU // NS + 1, _upd, None)
        plsc.subcore_barrier()


def _dif_step(z0, z1, src_p, alpha, dst_p):
    fn = pl.kernel(
        _dif_body,
        out_type=[
            jax.ShapeDtypeStruct((N, F2), jnp.float32),
            jax.ShapeDtypeStruct((N, F2), jnp.float32),
        ],
        mesh=_mesh(),
        scratch_types=[
            pltpu.VMEM((UR, F2), jnp.float32),
            pltpu.VMEM((DC,), jnp.int32),
            pltpu.VMEM((DC,), jnp.int32),
            pltpu.VMEM((DC,), jnp.int32),
            pltpu.VMEM((DC,), jnp.int32),
            pltpu.VMEM((DC,), jnp.int32),
            pltpu.VMEM((DC, L), jnp.float32),
            pltpu.VMEM((DC, L), jnp.float32),
            pltpu.VMEM((DC, F2), jnp.float32),
            pltpu.VMEM((DC, F2), jnp.float32),
            pltpu.VMEM((UR, F2), jnp.float32),
            pltpu.VMEM((UR, F2), jnp.float32),
            pltpu.VMEM_SHARED((SROWS, F2), jnp.float32),
            pltpu.SemaphoreType.DMA,
            pltpu.SemaphoreType.DMA,
        ],
    )
    return tuple(fn(z0, z1, src_p, alpha, dst_p))


def _elu_body(z_ref, o_ref):
    z = z_ref[...]
    o_ref[...] = jnp.where(z > 0, z, jnp.exp(z) - 1.0)


def _elu(z):
    return pl.pallas_call(
        _elu_body,
        grid=(GN,),
        in_specs=[pl.BlockSpec((BN, F), lambda i: (i, 0))],
        out_specs=pl.BlockSpec((BN, F), lambda i: (i, 0)),
        out_shape=jax.ShapeDtypeStruct((N, F), jnp.float32),
    )(z)


def _decoder_body(fi_ref, fj_ref, o_ref):
    o_ref[...] = lax.dot_general(
        fi_ref[...], fj_ref[...],
        (((1,), (1,)), ((), ())),
        preferred_element_type=jnp.float32)


def _decoder(f):
    return pl.pallas_call(
        _decoder_body,
        grid=(GN, GN),
        in_specs=[
            pl.BlockSpec((BN, F), lambda i, j: (i, 0)),
            pl.BlockSpec((BN, F), lambda i, j: (j, 0)),
        ],
        out_specs=pl.BlockSpec((BN, BN), lambda i, j: (i, j)),
        out_shape=jax.ShapeDtypeStruct((N, N), jnp.float32),
    )(f, f)


def kernel(x, edge_index, W, a_src, a_dst):
    src = edge_index[0]
    dst = edge_index[1]

    eye = jnp.eye(H, dtype=jnp.float32)
    Bs = (eye[:, None, :] * a_src[:, :, None]).reshape(F, H)
    Bd = (eye[:, None, :] * a_dst[:, :, None]).reshape(F, H)
    A8 = jnp.concatenate([Bs, Bd], axis=1)

    s8, smax = _scores(x, W, A8)
    m4 = smax[0, :H] + smax[0, H:]
    m4 = jnp.where(m4 > 0, m4, 0.2 * m4)
    mtab = jnp.pad(m4, (0, L - H))[None, :]

    sa = jnp.pad(s8[:, :H], ((0, 0), (0, GW - H)))
    sb = jnp.pad(s8[:, H:], ((0, 0), (0, GW - H)))

    pad = jnp.zeros((EPAD - E,), jnp.int32)
    src_p = jnp.concatenate([src, pad])
    dst_p = jnp.concatenate([dst, pad])

    exT, dpart = _edge_pass(src_p, dst_p, sa, sb, mtab)
    dsum = _dsum(dpart)
    dg = _dgather(dst_p, dsum)
    alpha = _alpha_tc(exT, dg, (dst_p % 4)[:, None])
    def _step(_, zz):
        return _dif_step(zz[0], zz[1], src_p, alpha, dst_p)

    z0, z1 = lax.fori_loop(0, STEPS, _step, (x[:, :F2], x[:, F2:]))
    z = jnp.concatenate([z0, z1], axis=1)

    f = _elu(z)
    recon = _decoder(f)
    return (f, recon, x, z)


# speedup vs baseline: 1.2970x; 1.0140x over previous
import jax
import jax.numpy as jnp
from jax import lax
from jax.experimental import pallas as pl
from jax.experimental.pallas import tpu as pltpu
from jax.experimental.pallas import tpu_sc as plsc

N = 10000
F = 256
H = 4
E = 160000
STEPS = 4
DT = 0.25

NC = 2
NS = 16
NW = NC * NS
L = 16

EPAD = 163840
EW = EPAD // NW
CH = 128
NCH = EW // CH

ETILE = EPAD // NS
DC = 128
NDC = ETILE // DC

NHALF = N // NC
SROWS = 5120
DUMMY = NHALF

DN = 10240
DTILE = DN // NS
DNF = DN * H

BN = 512
GN = (N + BN - 1) // BN

GW = 128


def _mesh():
    return plsc.VectorSubcoreMesh(core_axis_name="c", subcore_axis_name="s",
                                  num_cores=NC, num_subcores=NS)


def _scores_body(x_ref, w_ref, a_ref, s_ref, m_ref):
    i = pl.program_id(0)
    h = jnp.dot(x_ref[...], w_ref[...], preferred_element_type=jnp.float32)
    s = jnp.dot(h, a_ref[...], preferred_element_type=jnp.float32)
    s_ref[...] = s
    rows = i * BN + lax.broadcasted_iota(jnp.int32, s.shape, 0)
    sm = jnp.where(rows < N, s, -jnp.inf)
    bm = jnp.max(sm, axis=0, keepdims=True)

    @pl.when(i == 0)
    def _():
        m_ref[...] = bm

    @pl.when(i > 0)
    def _():
        m_ref[...] = jnp.maximum(m_ref[...], bm)


def _scores(x, W, A8):
    return pl.pallas_call(
        _scores_body,
        grid=(GN,),
        in_specs=[
            pl.BlockSpec((BN, F), lambda i: (i, 0)),
            pl.BlockSpec((F, F), lambda i: (0, 0)),
            pl.BlockSpec((F, 2 * H), lambda i: (0, 0)),
        ],
        out_specs=[
            pl.BlockSpec((BN, 2 * H), lambda i: (i, 0)),
            pl.BlockSpec((1, 2 * H), lambda i: (0, 0)),
        ],
        out_shape=[
            jax.ShapeDtypeStruct((N, 2 * H), jnp.float32),
            jax.ShapeDtypeStruct((1, 2 * H), jnp.float32),
        ],
    )(x, W, A8)


def _edge_body(src_ref, dst_ref, sa_ref, sb_ref, m_ref, ex_ref, dpart_ref,
               mv, isrc, idst, rsrc, rdst, exv, dnf, sem):
    cid = lax.axis_index("c")
    sid = lax.axis_index("s")
    wid = cid * NS + sid

    def _zr(r, c):
        dnf[pl.ds(r * L, L)] = jnp.zeros((L,), jnp.float32)
        return c
    lax.fori_loop(0, DNF // L, _zr, None)

    pltpu.sync_copy(m_ref, mv)

    iota = lax.iota(jnp.int32, L)
    pat = jnp.where(iota < H, 1.0, 0.0)
    mvec = mv[0, :]

    def _chunk(k, c):
        base = wid * EW + k * CH
        pltpu.sync_copy(src_ref.at[pl.ds(base, CH)], isrc)
        pltpu.sync_copy(dst_ref.at[pl.ds(base, CH)], idst)
        pltpu.async_copy(sa_ref.at[isrc], rsrc, sem).wait()
        pltpu.async_copy(sb_ref.at[idst], rdst, sem).wait()
        for e in range(CH):
            t = rsrc[e, pl.ds(0, L)] + rdst[e, pl.ds(0, L)]
            lr = jnp.where(t > 0, t, 0.2 * t)
            exv[e, :] = jnp.exp(lr - mvec) * pat
        pltpu.sync_copy(exv, ex_ref.at[pl.ds(base, CH)])

        @pl.when(base < E)
        def _():
            for v in range(CH // L):
                dl = idst[pl.ds(v * L, L)]
                for i in range(L):
                    e = v * L + i
                    plsc.addupdate(dnf.at[pl.ds(dl[i] * H, L)], exv[e, :])
        return c

    lax.fori_loop(0, NCH, _chunk, None)

    pltpu.sync_copy(dnf, dpart_ref.at[wid, pl.ds(0, DNF)])


def _edge_pass(src_p, dst_p, sa, sb, mtab):
    fn = pl.kernel(
        _edge_body,
        out_type=[
            jax.ShapeDtypeStruct((EPAD, L), jnp.float32),
            jax.ShapeDtypeStruct((NW, 2 * DNF), jnp.float32),
        ],
        mesh=_mesh(),
        scratch_types=[
            pltpu.VMEM((1, L), jnp.float32),
            pltpu.VMEM((CH,), jnp.int32),
            pltpu.VMEM((CH,), jnp.int32),
            pltpu.VMEM((CH, GW), jnp.float32),
            pltpu.VMEM((CH, GW), jnp.float32),
            pltpu.VMEM((CH, L), jnp.float32),
            pltpu.VMEM((DNF,), jnp.float32),
            pltpu.SemaphoreType.DMA,
        ],
    )
    return fn(src_p, dst_p, sa, sb, mtab)


def _dsum_body(dpart_ref, dsum_ref, acc, tmp, tw):
    cid = lax.axis_index("c")
    sid = lax.axis_index("s")
    wid = cid * NS + sid
    W0 = DNF // NW
    R0 = DN // NW

    pltpu.sync_copy(dpart_ref.at[0, pl.ds(wid * W0, W0)], acc.at[pl.ds(0, W0)])

    def _zt(r, c):
        acc[pl.ds(W0 + r * L, L)] = jnp.zeros((L,), jnp.float32)
        return c
    lax.fori_loop(0, (acc.shape[0] - W0) // L, _zt, None)

    for w in range(1, NW):
        pltpu.sync_copy(dpart_ref.at[w, pl.ds(wid * W0, W0)], tmp)

        def _add(v, c):
            acc[pl.ds(v * L, L)] = acc[pl.ds(v * L, L)] + tmp[pl.ds(v * L, L)]
            return c
        lax.fori_loop(0, W0 // L, _add, None)

    G0 = R0 // 4

    def _zw(r, c):
        for j in range(GW // L):
            tw[r, pl.ds(j * L, L)] = jnp.zeros((L,), jnp.float32)
        return c
    lax.fori_loop(0, G0, _zw, None)

    def _w(r, c):
        tw[r, pl.ds(0, L)] = acc[pl.ds(r * L, L)]
        return c
    lax.fori_loop(0, G0, _w, None)
    pltpu.sync_copy(tw, dsum_ref.at[pl.ds(wid * G0, G0)])


def _dsum(dpart):
    fn = pl.kernel(
        _dsum_body,
        out_type=jax.ShapeDtypeStruct((DN // 4, GW), jnp.float32),
        mesh=_mesh(),
        scratch_types=[
            pltpu.VMEM((DNF // NW + L, ), jnp.float32),
            pltpu.VMEM((DNF // NW,), jnp.float32),
            pltpu.VMEM((DN // NW // 4, GW), jnp.float32),
        ],
    )
    return fn(dpart)


def _dgather_body(dst_ref, dsum_ref, dg_ref, idst, gix, db, g16, sem):
    cid = lax.axis_index("c")
    sid = lax.axis_index("s")
    wid = cid * NS + sid

    def _chunk(k, c):
        base = wid * EW + k * CH
        pltpu.sync_copy(dst_ref.at[pl.ds(base, CH)], idst)
        for v in range(CH // L):
            gix[pl.ds(v * L, L)] = lax.shift_right_logical(
                idst[pl.ds(v * L, L)], 2)
        pltpu.async_copy(dsum_ref.at[gix], db, sem).wait()
        for e in range(CH):
            g16[e, :] = db[e, pl.ds(0, L)]
        pltpu.sync_copy(g16, dg_ref.at[pl.ds(base, CH)])
        return c

    lax.fori_loop(0, NCH, _chunk, None)


def _dgather(dst_p, dsum):
    fn = pl.kernel(
        _dgather_body,
        out_type=jax.ShapeDtypeStruct((EPAD, L), jnp.float32),
        mesh=_mesh(),
        scratch_types=[
            pltpu.VMEM((CH,), jnp.int32),
            pltpu.VMEM((CH,), jnp.int32),
            pltpu.VMEM((CH, GW), jnp.float32),
            pltpu.VMEM((CH, L), jnp.float32),
            pltpu.SemaphoreType.DMA,
        ],
    )
    return fn(dst_p, dsum)


ABN = 2048


def _alpha_tc_body(ex_ref, dg_ref, dm_ref, a_ref):
    i = pl.program_id(0)
    m = dm_ref[...]
    d4 = jnp.zeros((ABN, H), jnp.float32)
    for k in range(4):
        d4 = jnp.where(m == k, dg_ref[:, 4 * k:4 * k + 4], d4)
    acc = ex_ref[:, :H] / (d4 + 1e-16)
    s = jnp.sum(acc, axis=1, keepdims=True) * (1.0 / H)
    rows = i * ABN + lax.broadcasted_iota(jnp.int32, s.shape, 0)
    s = jnp.where(rows < E, s, 0.0)
    a_ref[...] = jnp.broadcast_to(s, (ABN, L))


def _alpha_tc(exT, dg, dm):
    return pl.pallas_call(
        _alpha_tc_body,
        grid=(EPAD // ABN,),
        in_specs=[
            pl.BlockSpec((ABN, L), lambda i: (i, 0)),
            pl.BlockSpec((ABN, L), lambda i: (i, 0)),
            pl.BlockSpec((ABN, 1), lambda i: (i, 0)),
        ],
        out_specs=pl.BlockSpec((ABN, L), lambda i: (i, 0)),
        out_shape=jax.ShapeDtypeStruct((EPAD, L), jnp.float32),
    )(exT, dg, dm)


F2 = F // 2
FJ = F2 // L
UR = 40
NU = NHALF // UR


def _dif_body(z0_ref, z1_ref, src_ref, alpha_ref, dst_ref,
              zo0_ref, zo1_ref,
              zb2, isrc_a, isrc_b, idst_a, idst_b, sidx_a, sidx_b,
              ab_a, ab_b, rows_a, rows_b, uagg, uz, agg_sh,
              sem_a, sem_b, ssem_a, ssem_b):
    cid = lax.axis_index("c")
    sid = lax.axis_index("s")
    zin = [z0_ref, z1_ref]
    zout = [zo0_ref, zo1_ref]
    bufs = ((isrc_a, idst_a, ab_a, rows_a, sidx_a, sem_a, ssem_a),
            (isrc_b, idst_b, ab_b, rows_b, sidx_b, sem_b, ssem_b))

    for r in range(UR):
        for j in range(FJ):
            zb2[r, pl.ds(j * L, L)] = jnp.zeros((L,), jnp.float32)

    def _fetch(zr, k, bset):
        isrc, idst, ab, rows, sidx, sem, ssem = bset
        base = sid * ETILE + k * DC
        pltpu.async_copy(src_ref.at[pl.ds(base, DC)], isrc, sem)

    def _fetch2(zr, k, bset):
        isrc, idst, ab, rows, sidx, sem, ssem = bset
        base = sid * ETILE + k * DC
        pltpu.make_async_copy(src_ref.at[pl.ds(0, DC)], isrc, sem).wait()

        @pl.when(k >= 2)
        def _():
            pltpu.make_async_copy(rows, agg_sh.at[sidx], ssem).wait()
        pltpu.async_copy(zr.at[isrc], rows, sem)
        pltpu.async_copy(dst_ref.at[pl.ds(base, DC)], idst, sem)
        pltpu.async_copy(alpha_ref.at[pl.ds(base, DC)], ab, sem)

    def _consume(zr, k, bset):
        isrc, idst, ab, rows, sidx, sem, ssem = bset
        pltpu.make_async_copy(dst_ref.at[pl.ds(0, DC)], idst, sem).wait()
        pltpu.make_async_copy(alpha_ref.at[pl.ds(0, DC)], ab, sem).wait()
        pltpu.make_async_copy(zr.at[isrc], rows, sem).wait()
        for e in range(DC):
            asp = ab[e, :]
            for j in range(FJ):
                rows[e, pl.ds(j * L, L)] = rows[e, pl.ds(j * L, L)] * asp
        for v in range(DC // L):
            dl = idst[pl.ds(v * L, L)]
            loc = dl - cid * NHALF
            ok = (loc >= 0) & (loc < NHALF)
            sidx[pl.ds(v * L, L)] = jnp.where(ok, loc, DUMMY)
        pltpu.async_copy(rows, agg_sh.at[sidx], ssem, add=True)

    for half in range(2):
        for u in range(SROWS // NS // UR):
            pltpu.sync_copy(
                zb2, agg_sh.at[pl.ds((sid * (SROWS // NS // UR) + u) * UR,
                                     UR)])
        plsc.subcore_barrier()

        zr = zin[half]
        _fetch(zr, 0, bufs[0])
        _fetch2(zr, 0, bufs[0])

        def _pair(k2, c):
            for b in range(2):
                k = k2 * 2 + b

                @pl.when(k + 1 < NDC)
                def _():
                    _fetch(zr, k + 1, bufs[1 - b])
                    _fetch2(zr, k + 1, bufs[1 - b])
                _consume(zr, k, bufs[b])
            return c

        lax.fori_loop(0, NDC // 2, _pair, None)
        pltpu.make_async_copy(rows_a, agg_sh.at[sidx_a], ssem_a).wait()
        pltpu.make_async_copy(rows_b, agg_sh.at[sidx_b], ssem_b).wait()
        plsc.subcore_barrier()

        def _upd(u, c):
            idx = sid * (NU // NS + 1) + u

            @pl.when(idx < NU)
            def _():
                g = cid * NHALF + idx * UR
                pltpu.sync_copy(agg_sh.at[pl.ds(idx * UR, UR)], uagg)
                pltpu.sync_copy(zr.at[pl.ds(g, UR)], uz)

                def _row(r, c2):
                    for j in range(FJ):
                        uz[r, pl.ds(j * L, L)] = (
                            uz[r, pl.ds(j * L, L)] * (1.0 - DT)
                            + uagg[r, pl.ds(j * L, L)] * DT)
                    return c2
                lax.fori_loop(0, UR, _row, None)
                pltpu.sync_copy(uz, zout[half].at[pl.ds(g, UR)])
            return c

        lax.fori_loop(0, NU // NS + 1, _upd, None)
        plsc.subcore_barrier()


def _dif_step(z0, z1, src_p, alpha, dst_p):
    fn = pl.kernel(
        _dif_body,
        out_type=[
            jax.ShapeDtypeStruct((N, F2), jnp.float32),
            jax.ShapeDtypeStruct((N, F2), jnp.float32),
        ],
        mesh=_mesh(),
        scratch_types=[
            pltpu.VMEM((UR, F2), jnp.float32),
            pltpu.VMEM((DC,), jnp.int32),
            pltpu.VMEM((DC,), jnp.int32),
            pltpu.VMEM((DC,), jnp.int32),
            pltpu.VMEM((DC,), jnp.int32),
            pltpu.VMEM((DC,), jnp.int32),
            pltpu.VMEM((DC,), jnp.int32),
            pltpu.VMEM((DC, L), jnp.float32),
            pltpu.VMEM((DC, L), jnp.float32),
            pltpu.VMEM((DC, F2), jnp.float32),
            pltpu.VMEM((DC, F2), jnp.float32),
            pltpu.VMEM((UR, F2), jnp.float32),
            pltpu.VMEM((UR, F2), jnp.float32),
            pltpu.VMEM_SHARED((SROWS, F2), jnp.float32),
            pltpu.SemaphoreType.DMA,
            pltpu.SemaphoreType.DMA,
            pltpu.SemaphoreType.DMA,
            pltpu.SemaphoreType.DMA,
        ],
    )
    return tuple(fn(z0, z1, src_p, alpha, dst_p))


def _elu_body(z_ref, o_ref):
    z = z_ref[...]
    o_ref[...] = jnp.where(z > 0, z, jnp.exp(z) - 1.0)


def _elu(z):
    return pl.pallas_call(
        _elu_body,
        grid=(GN,),
        in_specs=[pl.BlockSpec((BN, F), lambda i: (i, 0))],
        out_specs=pl.BlockSpec((BN, F), lambda i: (i, 0)),
        out_shape=jax.ShapeDtypeStruct((N, F), jnp.float32),
    )(z)


def _decoder_body(fi_ref, fj_ref, o_ref):
    o_ref[...] = lax.dot_general(
        fi_ref[...], fj_ref[...],
        (((1,), (1,)), ((), ())),
        preferred_element_type=jnp.float32)


def _decoder(f):
    return pl.pallas_call(
        _decoder_body,
        grid=(GN, GN),
        in_specs=[
            pl.BlockSpec((BN, F), lambda i, j: (i, 0)),
            pl.BlockSpec((BN, F), lambda i, j: (j, 0)),
        ],
        out_specs=pl.BlockSpec((BN, BN), lambda i, j: (i, j)),
        out_shape=jax.ShapeDtypeStruct((N, N), jnp.float32),
    )(f, f)


def kernel(x, edge_index, W, a_src, a_dst):
    src = edge_index[0]
    dst = edge_index[1]

    eye = jnp.eye(H, dtype=jnp.float32)
    Bs = (eye[:, None, :] * a_src[:, :, None]).reshape(F, H)
    Bd = (eye[:, None, :] * a_dst[:, :, None]).reshape(F, H)
    A8 = jnp.concatenate([Bs, Bd], axis=1)

    s8, smax = _scores(x, W, A8)
    m4 = smax[0, :H] + smax[0, H:]
    m4 = jnp.where(m4 > 0, m4, 0.2 * m4)
    mtab = jnp.pad(m4, (0, L - H))[None, :]

    sa = jnp.pad(s8[:, :H], ((0, 0), (0, GW - H)))
    sb = jnp.pad(s8[:, H:], ((0, 0), (0, GW - H)))

    pad = jnp.zeros((EPAD - E,), jnp.int32)
    src_p = jnp.concatenate([src, pad])
    dst_p = jnp.concatenate([dst, pad])

    exT, dpart = _edge_pass(src_p, dst_p, sa, sb, mtab)
    dsum = _dsum(dpart)
    dg = _dgather(dst_p, dsum)
    alpha = _alpha_tc(exT, dg, (dst_p % 4)[:, None])
    def _step(_, zz):
        return _dif_step(zz[0], zz[1], src_p, alpha, dst_p)

    z0, z1 = lax.fori_loop(0, STEPS, _step, (x[:, :F2], x[:, F2:]))
    z = jnp.concatenate([z0, z1], axis=1)

    f = _elu(z)
    recon = _decoder(f)
    return (f, recon, x, z)
